# baseline (speedup 1.0000x reference)
import jax
import jax.numpy as jnp
from jax.experimental import pallas as pl
from jax.experimental.pallas import tpu as pltpu

SEQ = 200
DIM = 64
BS = 8
BL = 1024


def _add_kernel(ex_ref, cat_ref, pos_ref, out_ref):
    out_ref[:] = ex_ref[:] + cat_ref[:] + pos_ref[:][:, :, None]


def kernel(exercises, categories, position_embed):
    B = exercises.shape[0]
    ex_t = jnp.transpose(exercises, (1, 2, 0))
    cat_t = jnp.transpose(categories, (1, 2, 0))
    out_t = pl.pallas_call(
        _add_kernel,
        grid=(SEQ // BS, B // BL),
        in_specs=[
            pl.BlockSpec((BS, DIM, BL), lambda i, j: (i, 0, j)),
            pl.BlockSpec((BS, DIM, BL), lambda i, j: (i, 0, j)),
            pl.BlockSpec((BS, DIM), lambda i, j: (i, 0)),
        ],
        out_specs=pl.BlockSpec((BS, DIM, BL), lambda i, j: (i, 0, j)),
        out_shape=jax.ShapeDtypeStruct((SEQ, DIM, B), jnp.float32),
        compiler_params=pltpu.CompilerParams(
            dimension_semantics=("arbitrary", "arbitrary"),
        ),
    )(ex_t, cat_t, position_embed)
    return jnp.transpose(out_t, (2, 0, 1))
